# baseline (speedup 1.0000x reference)
import functools
from math import pi

import numpy as np
import jax
import jax.numpy as jnp
from jax import lax
from jax.experimental import pallas as pl
from jax.experimental.pallas import tpu as pltpu
from jax.experimental.pallas import tpu_sc as plsc

B, NV, NF = 2, 25000, 22500
NUM_COORS, DIM_COOR = 512, 64
NUM_ANGLE, DIM_ANGLE = 128, 16
NUM_AREA, DIM_AREA = 128, 16
NUM_NORM, DIM_NORM = 128, 64
NUM_EMNO, DIM_EMNO = 128, 16
NUM_EMANG, DIM_EMANG = 128, 64
NUM_EMFREQ, DIM_EMFREQ = 512, 16
EPS = 1e-5

FB = 512
NBLK = -(-NF // FB)
NFP = NBLK * FB
ROWS = 3 * B * NFP
OUT_DIM = 1056


def _disc(t, lo, hi, n):
    t = (t - lo) / (hi - lo) * n - 0.5
    return jnp.clip(jnp.round(t), 0, n - 1).astype(jnp.int32)


def _l2norm(t, axis):
    nrm = jnp.sqrt(jnp.sum(t * t, axis=axis, keepdims=True))
    return t / jnp.maximum(nrm, 1e-12)


def _mono_lookup(s, table):
    f = s.shape[0]
    one = jnp.ones((f, 1), jnp.bfloat16)
    zero = jnp.zeros((f, 1), jnp.bfloat16)
    oh = (jnp.concatenate([one, s], axis=1)
          - jnp.concatenate([s, zero], axis=1))
    return jnp.dot(oh, table, preferred_element_type=jnp.float32)


def _ge_lookup(t_col, bounds, table):
    return _mono_lookup((t_col >= bounds).astype(jnp.bfloat16), table)


def _acos_lookup(x_col, th, table):
    return _mono_lookup((x_col <= th).astype(jnp.bfloat16), table)


def _tc_body(g0, g1, g2, ivec_r, freq_r, thr_r, bcoor_r, barea_r, bnorm_r,
             coor_t, angle_t, area_t, norm_t, emno_t, emang_t, emfreq_t,
             out_ref):
    thr = thr_r[...]
    bcoor = bcoor_r[...]
    barea = barea_r[...]
    bnorm = bnorm_r[...]
    v0 = g0[0, 0]
    v1 = g1[0, 0]
    v2 = g2[0, 0]
    c0 = v0[:, :3]
    c1 = v1[:, :3]
    c2 = v2[:, :3]

    e0 = c0 - c2
    e1 = c1 - c0
    e2 = c2 - c1

    n0 = _l2norm(e0, 1)
    n1 = _l2norm(e1, 1)
    n2 = _l2norm(e2, 1)

    def _rollc(t):
        return jnp.concatenate([t[:, 2:3], t[:, 0:2]], axis=1)

    normdot = -(n0 * _rollc(n0) + n1 * _rollc(n1) + n2 * _rollc(n2))
    normdot = jnp.clip(normdot, -1.0 + EPS, 1.0 - EPS)

    crx = e0[:, 1:2] * e1[:, 2:3] - e0[:, 2:3] * e1[:, 1:2]
    cry = e0[:, 2:3] * e1[:, 0:1] - e0[:, 0:1] * e1[:, 2:3]
    crz = e0[:, 0:1] * e1[:, 1:2] - e0[:, 1:2] * e1[:, 0:1]
    cr = jnp.concatenate([crx, cry, crz], axis=1)
    normals = _l2norm(cr, 1)
    area = jnp.sqrt(jnp.sum(cr * cr, axis=1, keepdims=True)) * 0.5

    iv = ivec_r[0]
    nv = _l2norm(iv, 0)
    ln = _l2norm(normals, 1)
    nd2 = -(ln[:, 0:1] * nv[0:1, :] + ln[:, 1:2] * nv[1:2, :]
            + ln[:, 2:3] * nv[2:3, :])
    nd2 = jnp.clip(nd2, -1.0 + EPS, 1.0 - EPS)

    parts = []
    for cc in (c0, c1, c2):
        for k in range(3):
            parts.append(_ge_lookup(cc[:, k:k + 1], bcoor, coor_t[...]))
    for k in range(3):
        parts.append(_acos_lookup(normdot[:, k:k + 1], thr, angle_t[...]))
    parts.append(_ge_lookup(area, barea, area_t[...]))
    for k in range(3):
        parts.append(_ge_lookup(normals[:, k:k + 1], bnorm, norm_t[...]))
    parts.append(_acos_lookup(nd2, thr, emno_t[...]))
    d_emang = _disc(iv, -1.0, 1.0, NUM_EMANG)
    oh_ang = (lax.broadcasted_iota(jnp.int32, (3, NUM_EMANG), 1)
              == d_emang).astype(jnp.bfloat16)
    emang_rows = jnp.dot(oh_ang, emang_t[...],
                         preferred_element_type=jnp.float32)
    for k in range(3):
        parts.append(jnp.broadcast_to(emang_rows[k:k + 1, :],
                                      (FB, DIM_EMANG)))
    d_fq = _disc(freq_r[0], 0.0, 1.0, NUM_EMFREQ)
    oh_fq = (lax.broadcasted_iota(jnp.int32, (1, NUM_EMFREQ), 1)
             == d_fq).astype(jnp.bfloat16)
    fq_row = jnp.dot(oh_fq, emfreq_t[...],
                     preferred_element_type=jnp.float32)
    parts.append(jnp.broadcast_to(fq_row, (FB, DIM_EMFREQ)))

    out_ref[0] = jnp.concatenate(parts, axis=1)


def _tc_call(gath, ivec3, freq3, thr, bcoor, barea, bnorm, coor_t, angle_t,
             area_t, norm_t, emno_t, emang_t, emfreq_t):
    full = lambda shape: pl.BlockSpec(shape, lambda b, i: (0, 0))
    slot = lambda j: pl.BlockSpec((1, 1, FB, 16),
                                  lambda b, i, j=j: (j, b, i, 0))
    return pl.pallas_call(
        _tc_body,
        grid=(B, NBLK),
        in_specs=[
            slot(0), slot(1), slot(2),
            pl.BlockSpec((1, 3, 1), lambda b, i: (b, 0, 0)),
            pl.BlockSpec((1, 1, 1), lambda b, i: (b, 0, 0)),
            full((1, NUM_ANGLE - 1)),
            full((1, NUM_COORS - 1)),
            full((1, NUM_AREA - 1)),
            full((1, NUM_NORM - 1)),
            full((NUM_COORS, DIM_COOR)),
            full((NUM_ANGLE, DIM_ANGLE)),
            full((NUM_AREA, DIM_AREA)),
            full((NUM_NORM, DIM_NORM)),
            full((NUM_EMNO, DIM_EMNO)),
            full((NUM_EMANG, DIM_EMANG)),
            full((NUM_EMFREQ, DIM_EMFREQ)),
        ],
        out_specs=pl.BlockSpec((1, FB, OUT_DIM), lambda b, i: (b, i, 0)),
        out_shape=jax.ShapeDtypeStruct((B, NF, OUT_DIM), jnp.float32),
        compiler_params=pltpu.CompilerParams(
            dimension_semantics=("parallel", "parallel")),
    )(gath, gath, gath, ivec3, freq3, thr, bcoor, barea, bnorm, coor_t,
      angle_t, area_t, norm_t, emno_t, emang_t, emfreq_t)


def _sc_gather(vtab, gidx):
    info = plsc.get_sparse_core_info()
    nw = info.num_cores * info.num_subcores
    rpw = ROWS // nw
    mesh = plsc.VectorSubcoreMesh(core_axis_name="c", subcore_axis_name="s")

    @functools.partial(
        pl.kernel, mesh=mesh,
        compiler_params=pltpu.CompilerParams(use_tc_tiling_on_sc=False),
        out_type=jax.ShapeDtypeStruct((ROWS, 16), jnp.float32),
        scratch_types=[
            pltpu.VMEM((rpw,), jnp.int32),
            pltpu.VMEM((rpw, 16), jnp.float32),
            pltpu.SemaphoreType.DMA,
        ],
    )
    def k(tab_hbm, idx_hbm, out_hbm, idx_v, rows_v, sem):
        wid = lax.axis_index("s") * info.num_cores + lax.axis_index("c")
        base = wid * rpw
        pltpu.sync_copy(idx_hbm.at[pl.ds(base, rpw)], idx_v)
        pltpu.async_copy(tab_hbm.at[idx_v], rows_v, sem).wait()
        pltpu.sync_copy(rows_v, out_hbm.at[pl.ds(base, rpw)])

    return k(vtab, gidx)


def kernel(vertices, faces, theta, phi, freq, coor_embed, angle_embed,
           area_embed, normal_embed, emnoangle_embed, emangle_embed,
           emfreq_embed):
    vtab = jnp.pad(vertices.reshape(B * NV, 3), ((0, 0), (0, 13)))
    fidx = faces + (jnp.arange(B, dtype=jnp.int32) * NV)[:, None, None]
    fidx = jnp.transpose(fidx, (2, 0, 1))
    fidx = jnp.pad(fidx, ((0, 0), (0, 0), (0, NFP - NF)))
    gath = _sc_gather(vtab, fidx.reshape(-1)).reshape(3, B, NFP, 16)

    th = jnp.deg2rad(theta)
    ph = jnp.deg2rad(phi)
    ivec = jnp.stack([jnp.sin(ph) * jnp.cos(th), jnp.sin(ph) * jnp.sin(th),
                      jnp.cos(ph)], axis=1)
    thr = jnp.asarray(np.cos(np.arange(1, NUM_ANGLE) * np.pi / NUM_ANGLE),
                      jnp.float32).reshape(1, NUM_ANGLE - 1)

    def bounds(lo, hi, n):
        return jnp.asarray(lo + np.arange(1, n) * (hi - lo) / n,
                           jnp.float32).reshape(1, n - 1)

    bf = jnp.bfloat16
    return _tc_call(gath, ivec.reshape(B, 3, 1), freq.reshape(B, 1, 1),
                    thr, bounds(-1.0, 1.0, NUM_COORS),
                    bounds(0.0, 4.0, NUM_AREA), bounds(-1.0, 1.0, NUM_NORM),
                    coor_embed.astype(bf), angle_embed.astype(bf),
                    area_embed.astype(bf), normal_embed.astype(bf),
                    emnoangle_embed.astype(bf), emangle_embed.astype(bf),
                    emfreq_embed.astype(bf))


# speedup vs baseline: 1.1322x; 1.1322x over previous
import functools
from math import pi

import numpy as np
import jax
import jax.numpy as jnp
from jax import lax
from jax.experimental import pallas as pl
from jax.experimental.pallas import tpu as pltpu
from jax.experimental.pallas import tpu_sc as plsc

B, NV, NF = 2, 25000, 22500
NUM_COORS, DIM_COOR = 512, 64
NUM_ANGLE, DIM_ANGLE = 128, 16
NUM_AREA, DIM_AREA = 128, 16
NUM_NORM, DIM_NORM = 128, 64
NUM_EMNO, DIM_EMNO = 128, 16
NUM_EMANG, DIM_EMANG = 128, 64
NUM_EMFREQ, DIM_EMFREQ = 512, 16
EPS = 1e-5

FB = 512
NBLK = -(-NF // FB)
NFP = NBLK * FB
ROWS = 3 * B * NFP
OUT_DIM = 1056


def _disc(t, lo, hi, n):
    t = (t - lo) / (hi - lo) * n - 0.5
    return jnp.clip(jnp.round(t), 0, n - 1).astype(jnp.int32)


def _l2norm(t, axis):
    nrm = jnp.sqrt(jnp.sum(t * t, axis=axis, keepdims=True))
    return t / jnp.maximum(nrm, 1e-12)


def _acos_bucket(x, th):
    cmp = (x <= th).astype(jnp.int32)
    return jnp.sum(cmp, axis=1, keepdims=True)


def _lookup(idx_col, table, n):
    oh = (lax.broadcasted_iota(jnp.int32, (idx_col.shape[0], n), 1)
          == idx_col).astype(jnp.bfloat16)
    return jnp.dot(oh, table, preferred_element_type=jnp.float32)


def _tc_body(g0, g1, g2, ivec_r, freq_r, thr_r, bcoor_r, barea_r, bnorm_r,
             coor_t, angle_t, area_t, norm_t, emno_t, emang_t, emfreq_t,
             out_ref):
    thr = thr_r[...]
    bcoor = bcoor_r[...]
    barea = barea_r[...]
    bnorm = bnorm_r[...]
    v0 = g0[0, 0]
    v1 = g1[0, 0]
    v2 = g2[0, 0]
    c0 = v0[:, :3]
    c1 = v1[:, :3]
    c2 = v2[:, :3]

    e0 = c0 - c2
    e1 = c1 - c0
    e2 = c2 - c1

    n0 = _l2norm(e0, 1)
    n1 = _l2norm(e1, 1)
    n2 = _l2norm(e2, 1)

    def _rollc(t):
        return jnp.concatenate([t[:, 2:3], t[:, 0:2]], axis=1)

    normdot = -(n0 * _rollc(n0) + n1 * _rollc(n1) + n2 * _rollc(n2))
    normdot = jnp.clip(normdot, -1.0 + EPS, 1.0 - EPS)

    crx = e0[:, 1:2] * e1[:, 2:3] - e0[:, 2:3] * e1[:, 1:2]
    cry = e0[:, 2:3] * e1[:, 0:1] - e0[:, 0:1] * e1[:, 2:3]
    crz = e0[:, 0:1] * e1[:, 1:2] - e0[:, 1:2] * e1[:, 0:1]
    cr = jnp.concatenate([crx, cry, crz], axis=1)
    normals = _l2norm(cr, 1)
    area = jnp.sqrt(jnp.sum(cr * cr, axis=1, keepdims=True)) * 0.5

    iv = ivec_r[0]
    nv = _l2norm(iv, 0)
    ln = _l2norm(normals, 1)
    nd2 = -(ln[:, 0:1] * nv[0:1, :] + ln[:, 1:2] * nv[1:2, :]
            + ln[:, 2:3] * nv[2:3, :])
    nd2 = jnp.clip(nd2, -1.0 + EPS, 1.0 - EPS)

    parts = []
    for cc in (c0, c1, c2):
        for k in range(3):
            parts.append(_lookup(_disc(cc[:, k:k + 1], -1.0, 1.0, NUM_COORS),
                                 coor_t[...], NUM_COORS))
    for k in range(3):
        parts.append(_lookup(_acos_bucket(normdot[:, k:k + 1], thr),
                             angle_t[...], NUM_ANGLE))
    parts.append(_lookup(_disc(area, 0.0, 4.0, NUM_AREA), area_t[...],
                         NUM_AREA))
    d_nrm = _disc(normals, -1.0, 1.0, NUM_NORM)
    for k in range(3):
        parts.append(_lookup(d_nrm[:, k:k + 1], norm_t[...], NUM_NORM))
    parts.append(_lookup(_acos_bucket(nd2, thr), emno_t[...], NUM_EMNO))
    d_emang = _disc(iv, -1.0, 1.0, NUM_EMANG)
    oh_ang = (lax.broadcasted_iota(jnp.int32, (3, NUM_EMANG), 1)
              == d_emang).astype(jnp.bfloat16)
    emang_rows = jnp.dot(oh_ang, emang_t[...],
                         preferred_element_type=jnp.float32)
    for k in range(3):
        parts.append(jnp.broadcast_to(emang_rows[k:k + 1, :],
                                      (FB, DIM_EMANG)))
    d_fq = _disc(freq_r[0], 0.0, 1.0, NUM_EMFREQ)
    oh_fq = (lax.broadcasted_iota(jnp.int32, (1, NUM_EMFREQ), 1)
             == d_fq).astype(jnp.bfloat16)
    fq_row = jnp.dot(oh_fq, emfreq_t[...],
                     preferred_element_type=jnp.float32)
    parts.append(jnp.broadcast_to(fq_row, (FB, DIM_EMFREQ)))

    out_ref[0] = jnp.concatenate(parts, axis=1)


def _tc_call(gath, ivec3, freq3, thr, bcoor, barea, bnorm, coor_t, angle_t,
             area_t, norm_t, emno_t, emang_t, emfreq_t):
    full = lambda shape: pl.BlockSpec(shape, lambda b, i: (0, 0))
    slot = lambda j: pl.BlockSpec((1, 1, FB, 16),
                                  lambda b, i, j=j: (j, b, i, 0))
    return pl.pallas_call(
        _tc_body,
        grid=(B, NBLK),
        in_specs=[
            slot(0), slot(1), slot(2),
            pl.BlockSpec((1, 3, 1), lambda b, i: (b, 0, 0)),
            pl.BlockSpec((1, 1, 1), lambda b, i: (b, 0, 0)),
            full((1, NUM_ANGLE - 1)),
            full((1, NUM_COORS - 1)),
            full((1, NUM_AREA - 1)),
            full((1, NUM_NORM - 1)),
            full((NUM_COORS, DIM_COOR)),
            full((NUM_ANGLE, DIM_ANGLE)),
            full((NUM_AREA, DIM_AREA)),
            full((NUM_NORM, DIM_NORM)),
            full((NUM_EMNO, DIM_EMNO)),
            full((NUM_EMANG, DIM_EMANG)),
            full((NUM_EMFREQ, DIM_EMFREQ)),
        ],
        out_specs=pl.BlockSpec((1, FB, OUT_DIM), lambda b, i: (b, i, 0)),
        out_shape=jax.ShapeDtypeStruct((B, NF, OUT_DIM), jnp.float32),
        compiler_params=pltpu.CompilerParams(
            dimension_semantics=("parallel", "parallel")),
    )(gath, gath, gath, ivec3, freq3, thr, bcoor, barea, bnorm, coor_t,
      angle_t, area_t, norm_t, emno_t, emang_t, emfreq_t)


def _sc_gather(vtab, gidx):
    info = plsc.get_sparse_core_info()
    nw = info.num_cores * info.num_subcores
    rpw = ROWS // nw
    mesh = plsc.VectorSubcoreMesh(core_axis_name="c", subcore_axis_name="s")

    @functools.partial(
        pl.kernel, mesh=mesh,
        compiler_params=pltpu.CompilerParams(use_tc_tiling_on_sc=False),
        out_type=jax.ShapeDtypeStruct((ROWS, 16), jnp.float32),
        scratch_types=[
            pltpu.VMEM((rpw,), jnp.int32),
            pltpu.VMEM((rpw, 16), jnp.float32),
            pltpu.SemaphoreType.DMA,
        ],
    )
    def k(tab_hbm, idx_hbm, out_hbm, idx_v, rows_v, sem):
        wid = lax.axis_index("s") * info.num_cores + lax.axis_index("c")
        base = wid * rpw
        pltpu.sync_copy(idx_hbm.at[pl.ds(base, rpw)], idx_v)
        pltpu.async_copy(tab_hbm.at[idx_v], rows_v, sem).wait()
        pltpu.sync_copy(rows_v, out_hbm.at[pl.ds(base, rpw)])

    return k(vtab, gidx)


def kernel(vertices, faces, theta, phi, freq, coor_embed, angle_embed,
           area_embed, normal_embed, emnoangle_embed, emangle_embed,
           emfreq_embed):
    vtab = jnp.pad(vertices.reshape(B * NV, 3), ((0, 0), (0, 13)))
    fidx = faces + (jnp.arange(B, dtype=jnp.int32) * NV)[:, None, None]
    fidx = jnp.transpose(fidx, (2, 0, 1))
    fidx = jnp.pad(fidx, ((0, 0), (0, 0), (0, NFP - NF)))
    gath = _sc_gather(vtab, fidx.reshape(-1)).reshape(3, B, NFP, 16)

    th = jnp.deg2rad(theta)
    ph = jnp.deg2rad(phi)
    ivec = jnp.stack([jnp.sin(ph) * jnp.cos(th), jnp.sin(ph) * jnp.sin(th),
                      jnp.cos(ph)], axis=1)
    thr = jnp.asarray(np.cos(np.arange(1, NUM_ANGLE) * np.pi / NUM_ANGLE),
                      jnp.float32).reshape(1, NUM_ANGLE - 1)

    def bounds(lo, hi, n):
        return jnp.asarray(lo + np.arange(1, n) * (hi - lo) / n,
                           jnp.float32).reshape(1, n - 1)

    bf = jnp.bfloat16
    return _tc_call(gath, ivec.reshape(B, 3, 1), freq.reshape(B, 1, 1),
                    thr, bounds(-1.0, 1.0, NUM_COORS),
                    bounds(0.0, 4.0, NUM_AREA), bounds(-1.0, 1.0, NUM_NORM),
                    coor_embed.astype(bf), angle_embed.astype(bf),
                    area_embed.astype(bf), normal_embed.astype(bf),
                    emnoangle_embed.astype(bf), emangle_embed.astype(bf),
                    emfreq_embed.astype(bf))


# speedup vs baseline: 1.1666x; 1.0304x over previous
import functools
from math import pi

import numpy as np
import jax
import jax.numpy as jnp
from jax import lax
from jax.experimental import pallas as pl
from jax.experimental.pallas import tpu as pltpu
from jax.experimental.pallas import tpu_sc as plsc

B, NV, NF = 2, 25000, 22500
NUM_COORS, DIM_COOR = 512, 64
NUM_ANGLE, DIM_ANGLE = 128, 16
NUM_AREA, DIM_AREA = 128, 16
NUM_NORM, DIM_NORM = 128, 64
NUM_EMNO, DIM_EMNO = 128, 16
NUM_EMANG, DIM_EMANG = 128, 64
NUM_EMFREQ, DIM_EMFREQ = 512, 16
EPS = 1e-5

FB = 1024
NBLK = -(-NF // FB)
NFP = NBLK * FB
ROWS = 3 * B * NFP
OUT_DIM = 1056


def _disc(t, lo, hi, n):
    t = (t - lo) / (hi - lo) * n - 0.5
    return jnp.clip(jnp.round(t), 0, n - 1).astype(jnp.int32)


def _l2norm(t, axis):
    nrm = jnp.sqrt(jnp.sum(t * t, axis=axis, keepdims=True))
    return t / jnp.maximum(nrm, 1e-12)


def _acos_bucket(x, th):
    cmp = (x <= th).astype(jnp.int32)
    return jnp.sum(cmp, axis=1, keepdims=True)


def _lookup(idx_col, table, n):
    oh = (lax.broadcasted_iota(jnp.int32, (idx_col.shape[0], n), 1)
          == idx_col).astype(jnp.bfloat16)
    return jnp.dot(oh, table, preferred_element_type=jnp.float32)


def _tc_body(g0, g1, g2, ivec_r, freq_r, thr_r, bcoor_r, barea_r, bnorm_r,
             coor_t, angle_t, area_t, norm_t, emno_t, emang_t, emfreq_t,
             out_ref):
    thr = thr_r[...]
    bcoor = bcoor_r[...]
    barea = barea_r[...]
    bnorm = bnorm_r[...]
    v0 = g0[0, 0]
    v1 = g1[0, 0]
    v2 = g2[0, 0]
    c0 = v0[:, :3]
    c1 = v1[:, :3]
    c2 = v2[:, :3]

    e0 = c0 - c2
    e1 = c1 - c0
    e2 = c2 - c1

    n0 = _l2norm(e0, 1)
    n1 = _l2norm(e1, 1)
    n2 = _l2norm(e2, 1)

    def _rollc(t):
        return jnp.concatenate([t[:, 2:3], t[:, 0:2]], axis=1)

    normdot = -(n0 * _rollc(n0) + n1 * _rollc(n1) + n2 * _rollc(n2))
    normdot = jnp.clip(normdot, -1.0 + EPS, 1.0 - EPS)

    crx = e0[:, 1:2] * e1[:, 2:3] - e0[:, 2:3] * e1[:, 1:2]
    cry = e0[:, 2:3] * e1[:, 0:1] - e0[:, 0:1] * e1[:, 2:3]
    crz = e0[:, 0:1] * e1[:, 1:2] - e0[:, 1:2] * e1[:, 0:1]
    cr = jnp.concatenate([crx, cry, crz], axis=1)
    normals = _l2norm(cr, 1)
    area = jnp.sqrt(jnp.sum(cr * cr, axis=1, keepdims=True)) * 0.5

    iv = ivec_r[0]
    nv = _l2norm(iv, 0)
    ln = _l2norm(normals, 1)
    nd2 = -(ln[:, 0:1] * nv[0:1, :] + ln[:, 1:2] * nv[1:2, :]
            + ln[:, 2:3] * nv[2:3, :])
    nd2 = jnp.clip(nd2, -1.0 + EPS, 1.0 - EPS)

    parts = []
    for cc in (c0, c1, c2):
        for k in range(3):
            parts.append(_lookup(_disc(cc[:, k:k + 1], -1.0, 1.0, NUM_COORS),
                                 coor_t[...], NUM_COORS))
    for k in range(3):
        parts.append(_lookup(_acos_bucket(normdot[:, k:k + 1], thr),
                             angle_t[...], NUM_ANGLE))
    parts.append(_lookup(_disc(area, 0.0, 4.0, NUM_AREA), area_t[...],
                         NUM_AREA))
    d_nrm = _disc(normals, -1.0, 1.0, NUM_NORM)
    for k in range(3):
        parts.append(_lookup(d_nrm[:, k:k + 1], norm_t[...], NUM_NORM))
    parts.append(_lookup(_acos_bucket(nd2, thr), emno_t[...], NUM_EMNO))
    d_emang = _disc(iv, -1.0, 1.0, NUM_EMANG)
    oh_ang = (lax.broadcasted_iota(jnp.int32, (3, NUM_EMANG), 1)
              == d_emang).astype(jnp.bfloat16)
    emang_rows = jnp.dot(oh_ang, emang_t[...],
                         preferred_element_type=jnp.float32)
    for k in range(3):
        parts.append(jnp.broadcast_to(emang_rows[k:k + 1, :],
                                      (FB, DIM_EMANG)))
    d_fq = _disc(freq_r[0], 0.0, 1.0, NUM_EMFREQ)
    oh_fq = (lax.broadcasted_iota(jnp.int32, (1, NUM_EMFREQ), 1)
             == d_fq).astype(jnp.bfloat16)
    fq_row = jnp.dot(oh_fq, emfreq_t[...],
                     preferred_element_type=jnp.float32)
    parts.append(jnp.broadcast_to(fq_row, (FB, DIM_EMFREQ)))

    out_ref[0] = jnp.concatenate(parts, axis=1)


def _tc_call(gath, ivec3, freq3, thr, bcoor, barea, bnorm, coor_t, angle_t,
             area_t, norm_t, emno_t, emang_t, emfreq_t):
    full = lambda shape: pl.BlockSpec(shape, lambda b, i: (0, 0))
    slot = lambda j: pl.BlockSpec((1, 1, FB, 16),
                                  lambda b, i, j=j: (j, b, i, 0))
    return pl.pallas_call(
        _tc_body,
        grid=(B, NBLK),
        in_specs=[
            slot(0), slot(1), slot(2),
            pl.BlockSpec((1, 3, 1), lambda b, i: (b, 0, 0)),
            pl.BlockSpec((1, 1, 1), lambda b, i: (b, 0, 0)),
            full((1, NUM_ANGLE - 1)),
            full((1, NUM_COORS - 1)),
            full((1, NUM_AREA - 1)),
            full((1, NUM_NORM - 1)),
            full((NUM_COORS, DIM_COOR)),
            full((NUM_ANGLE, DIM_ANGLE)),
            full((NUM_AREA, DIM_AREA)),
            full((NUM_NORM, DIM_NORM)),
            full((NUM_EMNO, DIM_EMNO)),
            full((NUM_EMANG, DIM_EMANG)),
            full((NUM_EMFREQ, DIM_EMFREQ)),
        ],
        out_specs=pl.BlockSpec((1, FB, OUT_DIM), lambda b, i: (b, i, 0)),
        out_shape=jax.ShapeDtypeStruct((B, NF, OUT_DIM), jnp.float32),
        compiler_params=pltpu.CompilerParams(
            dimension_semantics=("parallel", "parallel")),
    )(gath, gath, gath, ivec3, freq3, thr, bcoor, barea, bnorm, coor_t,
      angle_t, area_t, norm_t, emno_t, emang_t, emfreq_t)


def _sc_gather(vtab, gidx):
    info = plsc.get_sparse_core_info()
    nw = info.num_cores * info.num_subcores
    rpw = ROWS // nw
    mesh = plsc.VectorSubcoreMesh(core_axis_name="c", subcore_axis_name="s")

    @functools.partial(
        pl.kernel, mesh=mesh,
        compiler_params=pltpu.CompilerParams(use_tc_tiling_on_sc=False),
        out_type=jax.ShapeDtypeStruct((ROWS, 16), jnp.float32),
        scratch_types=[
            pltpu.VMEM((rpw,), jnp.int32),
            pltpu.VMEM((rpw, 16), jnp.float32),
            pltpu.SemaphoreType.DMA,
        ],
    )
    def k(tab_hbm, idx_hbm, out_hbm, idx_v, rows_v, sem):
        wid = lax.axis_index("s") * info.num_cores + lax.axis_index("c")
        base = wid * rpw
        pltpu.sync_copy(idx_hbm.at[pl.ds(base, rpw)], idx_v)
        pltpu.async_copy(tab_hbm.at[idx_v], rows_v, sem).wait()
        pltpu.sync_copy(rows_v, out_hbm.at[pl.ds(base, rpw)])

    return k(vtab, gidx)


def kernel(vertices, faces, theta, phi, freq, coor_embed, angle_embed,
           area_embed, normal_embed, emnoangle_embed, emangle_embed,
           emfreq_embed):
    vtab = jnp.pad(vertices.reshape(B * NV, 3), ((0, 0), (0, 13)))
    fidx = faces + (jnp.arange(B, dtype=jnp.int32) * NV)[:, None, None]
    fidx = jnp.transpose(fidx, (2, 0, 1))
    fidx = jnp.pad(fidx, ((0, 0), (0, 0), (0, NFP - NF)))
    gath = _sc_gather(vtab, fidx.reshape(-1)).reshape(3, B, NFP, 16)

    th = jnp.deg2rad(theta)
    ph = jnp.deg2rad(phi)
    ivec = jnp.stack([jnp.sin(ph) * jnp.cos(th), jnp.sin(ph) * jnp.sin(th),
                      jnp.cos(ph)], axis=1)
    thr = jnp.asarray(np.cos(np.arange(1, NUM_ANGLE) * np.pi / NUM_ANGLE),
                      jnp.float32).reshape(1, NUM_ANGLE - 1)

    def bounds(lo, hi, n):
        return jnp.asarray(lo + np.arange(1, n) * (hi - lo) / n,
                           jnp.float32).reshape(1, n - 1)

    bf = jnp.bfloat16
    return _tc_call(gath, ivec.reshape(B, 3, 1), freq.reshape(B, 1, 1),
                    thr, bounds(-1.0, 1.0, NUM_COORS),
                    bounds(0.0, 4.0, NUM_AREA), bounds(-1.0, 1.0, NUM_NORM),
                    coor_embed.astype(bf), angle_embed.astype(bf),
                    area_embed.astype(bf), normal_embed.astype(bf),
                    emnoangle_embed.astype(bf), emangle_embed.astype(bf),
                    emfreq_embed.astype(bf))
